# baseline (speedup 1.0000x reference)
import jax
import jax.numpy as jnp
from jax.experimental import pallas as pl
from jax.experimental.pallas import tpu as pltpu

_BLOCK_ROWS = 120


def _attention_combine(ps, a):
    cols = [jnp.dot(p, a[:, s:s + 1], preferred_element_type=jnp.float32)
            for s, p in enumerate(ps)]
    scores = jnp.concatenate(cols, axis=1)
    scores = jnp.where(scores >= 0, scores, 0.2 * scores)
    m = jnp.max(scores, axis=1, keepdims=True)
    e = jnp.exp(scores - m)
    alpha = e / jnp.sum(e, axis=1, keepdims=True)
    out = ps[0] * alpha[:, 0:1]
    for s in range(1, 4):
        out = out + ps[s] * alpha[:, s:s + 1]
    return jnp.maximum(out, 0.0)


def _proj_body(x_ref, W_ref, o_ref):
    o_ref[...] = jnp.dot(x_ref[...], W_ref[...],
                         preferred_element_type=jnp.float32)


def _prop4(mats, hp):
    cat = jnp.concatenate([m[...] for m in mats], axis=0)
    pcat = jax.lax.dot_general(cat, hp, (((1,), (0,)), ((), ())),
                               preferred_element_type=jnp.float32)
    return [pcat[s * _BLOCK_ROWS:(s + 1) * _BLOCK_ROWS] for s in range(4)]


def _body(hp1_ref, A_ref, s1_ref, s2_ref, s3_ref, a1_ref,
          W2_ref, a2_ref, out_ref, hp2_ref):
    p = pl.program_id(0)
    i = pl.program_id(1)
    mats = (A_ref, s1_ref, s2_ref, s3_ref)
    n = hp1_ref.shape[0]

    @pl.when(p == 0)
    def _layer1():
        ps = _prop4(mats, hp1_ref[...])
        h1_blk = _attention_combine(ps, a1_ref[...])
        hp2_ref[pl.ds(i * _BLOCK_ROWS, _BLOCK_ROWS), :] = jnp.dot(
            h1_blk, W2_ref[...], preferred_element_type=jnp.float32)

    @pl.when(p == 1)
    def _layer2():
        ps = _prop4(mats, hp2_ref[:n, :])
        out = _attention_combine(ps, a2_ref[...])
        mx = jnp.max(out, axis=1, keepdims=True)
        shifted = out - mx
        lse = jnp.log(jnp.sum(jnp.exp(shifted), axis=1, keepdims=True))
        out_ref[...] = shifted - lse


def kernel(x, A_tilde, s1_sct, s2_sct, s3_sct, W1, a1, W2, a2):
    N, NFEAT = x.shape
    HID = W1.shape[1]
    NCLASS = W2.shape[1]
    R = pl.cdiv(N, _BLOCK_ROWS)

    hp1 = pl.pallas_call(
        _proj_body,
        out_shape=jax.ShapeDtypeStruct((N, HID), jnp.float32),
    )(x, W1)

    mat_spec = pl.BlockSpec((_BLOCK_ROWS, N), lambda p, i: (i, 0))

    def full(shape):
        return pl.BlockSpec(shape, lambda p, i: (0, 0))

    out = pl.pallas_call(
        _body,
        grid=(2, R),
        in_specs=[full((N, HID)), mat_spec, mat_spec, mat_spec, mat_spec,
                  full((HID, 4)), full((HID, NCLASS)), full((NCLASS, 4))],
        out_specs=pl.BlockSpec((_BLOCK_ROWS, NCLASS),
                               lambda p, i: (jnp.where(p == 0, R, i), 0)),
        out_shape=jax.ShapeDtypeStruct(((R + 1) * _BLOCK_ROWS, NCLASS),
                                       jnp.float32),
        scratch_shapes=[pltpu.VMEM((R * _BLOCK_ROWS, NCLASS), jnp.float32)],
        compiler_params=pltpu.CompilerParams(
            dimension_semantics=("arbitrary", "arbitrary")),
    )(hp1, A_tilde, s1_sct, s2_sct, s3_sct, a1, W2, a2)
    return out[:N]


# speedup vs baseline: 1.0096x; 1.0096x over previous
import jax
import jax.numpy as jnp
from jax.experimental import pallas as pl
from jax.experimental.pallas import tpu as pltpu

_BLOCK_ROWS = 136


def _attention_combine(ps, a):
    cols = [jnp.dot(p, a[:, s:s + 1], preferred_element_type=jnp.float32)
            for s, p in enumerate(ps)]
    scores = jnp.concatenate(cols, axis=1)
    scores = jnp.where(scores >= 0, scores, 0.2 * scores)
    m = jnp.max(scores, axis=1, keepdims=True)
    e = jnp.exp(scores - m)
    alpha = e / jnp.sum(e, axis=1, keepdims=True)
    out = ps[0] * alpha[:, 0:1]
    for s in range(1, 4):
        out = out + ps[s] * alpha[:, s:s + 1]
    return jnp.maximum(out, 0.0)


def _prop4(mats, hp):
    cat = jnp.concatenate([m[...] for m in mats], axis=0)
    pcat = jax.lax.dot_general(cat, hp, (((1,), (0,)), ((), ())),
                               preferred_element_type=jnp.float32)
    return [pcat[s * _BLOCK_ROWS:(s + 1) * _BLOCK_ROWS] for s in range(4)]


def _body(x_ref, A_ref, s1_ref, s2_ref, s3_ref, W1_ref, a1_ref,
          W2_ref, a2_ref, out_ref, hp1_ref, hp2_ref):
    p = pl.program_id(0)
    i = pl.program_id(1)
    mats = (A_ref, s1_ref, s2_ref, s3_ref)
    n = hp1_ref.shape[0]

    @pl.when(jnp.logical_and(p == 0, i == 0))
    def _project1():
        hp1_ref[...] = jnp.dot(x_ref[...], W1_ref[...],
                               preferred_element_type=jnp.float32)

    @pl.when(p == 0)
    def _layer1():
        ps = _prop4(mats, hp1_ref[...])
        h1_blk = _attention_combine(ps, a1_ref[...])
        hp2_ref[pl.ds(i * _BLOCK_ROWS, _BLOCK_ROWS), :] = jnp.dot(
            h1_blk, W2_ref[...], preferred_element_type=jnp.float32)

    @pl.when(p == 1)
    def _layer2():
        ps = _prop4(mats, hp2_ref[:n, :])
        out = _attention_combine(ps, a2_ref[...])
        mx = jnp.max(out, axis=1, keepdims=True)
        shifted = out - mx
        lse = jnp.log(jnp.sum(jnp.exp(shifted), axis=1, keepdims=True))
        out_ref[...] = shifted - lse


def kernel(x, A_tilde, s1_sct, s2_sct, s3_sct, W1, a1, W2, a2):
    N, NFEAT = x.shape
    HID = W1.shape[1]
    NCLASS = W2.shape[1]
    R = pl.cdiv(N, _BLOCK_ROWS)
    mat_spec = pl.BlockSpec((_BLOCK_ROWS, N), lambda p, i: (i, 0))

    def full(shape):
        return pl.BlockSpec(shape, lambda p, i: (0, 0))

    out = pl.pallas_call(
        _body,
        grid=(2, R),
        in_specs=[full((N, NFEAT)), mat_spec, mat_spec, mat_spec, mat_spec,
                  full((NFEAT, HID)), full((HID, 4)),
                  full((HID, NCLASS)), full((NCLASS, 4))],
        out_specs=pl.BlockSpec((_BLOCK_ROWS, NCLASS),
                               lambda p, i: (jnp.where(p == 0, R, i), 0)),
        out_shape=jax.ShapeDtypeStruct(((R + 1) * _BLOCK_ROWS, NCLASS),
                                       jnp.float32),
        scratch_shapes=[pltpu.VMEM((N, HID), jnp.float32),
                        pltpu.VMEM((R * _BLOCK_ROWS, NCLASS), jnp.float32)],
        compiler_params=pltpu.CompilerParams(
            dimension_semantics=("arbitrary", "arbitrary")),
    )(x, A_tilde, s1_sct, s2_sct, s3_sct, W1, a1, W2, a2)
    return out[:N]
